# baseline (speedup 1.0000x reference)
import functools
import jax
import jax.numpy as jnp
from jax import lax
from jax.experimental import pallas as pl
from jax.experimental.pallas import tpu as pltpu
from jax.experimental.pallas import tpu_sc as plsc

P = 24
DIM = 192
KS = 32
KP = 32
TOPK = 3
TEMP = 0.1
DEPTH = 4
BR = 24
RTOK = 1024
NEG = -3.0e38


def _conv_body(x_ref, bw_ref, bb_ref, pw_ref, pb_ref, o_ref):
    _, C, _, W = x_ref.shape
    Wp = W // P
    xb = x_ref[0].reshape(C, P * W)
    yb = jnp.dot(bw_ref[...], xb, preferred_element_type=jnp.float32)
    yb = yb + bb_ref[...]
    ybT = yb.astype(jnp.bfloat16).T
    t = (ybT.reshape(P, Wp, P, BR)
         .transpose(1, 0, 2, 3).reshape(Wp, P * P * BR))
    out = jnp.dot(t, pw_ref[...], preferred_element_type=jnp.float32) + pb_ref[...]
    o_ref[0, 0] = out


def _gelu(x):
    return 0.5 * x * (1.0 + jax.lax.erf(x * (2.0 ** -0.5)))


def _ln(x, s, b):
    m = x.mean(-1, keepdims=True)
    v = ((x - m) ** 2).mean(-1, keepdims=True)
    return (x - m) / jnp.sqrt(v + 1e-5) * s + b


def _router_body(t_ref, rw1_ref, rb1_ref, rw2_ref, rb2_ref, o_ref):
    h = jnp.dot(t_ref[...], rw1_ref[...], preferred_element_type=jnp.float32) + rb1_ref[...]
    h = _gelu(h)
    o_ref[...] = jnp.dot(h, rw2_ref[...], preferred_element_type=jnp.float32) + rb2_ref[...]


def _route_sc(logits_hbm, w_hbm, chunk_v, out_v):
    wid = lax.axis_index("s") * 2 + lax.axis_index("c")
    base = wid * 32
    pltpu.sync_copy(logits_hbm.at[pl.ds(base, 32)], chunk_v)
    zero = jnp.zeros((16,), jnp.float32)
    for r in range(32):
        for c in range(4):
            out_v[r, pl.ds(c * 16, 16)] = zero
    lane = lax.iota(jnp.int32, 16)
    for g in range(2):
        row_idx = lane + (g * 16)
        vecs = [plsc.load_gather(chunk_v, [row_idx, jnp.full((16,), d, jnp.int32)])
                for d in range(KS + KP)]
        for off in (0, KS):
            cur = [vecs[off + d] for d in range(KS)]
            tops, sels = [], []
            for _ in range(TOPK):
                m = cur[0]
                for d in range(1, KS):
                    m = jnp.maximum(m, cur[d])
                sel = jnp.full((16,), KS, jnp.int32)
                for d in reversed(range(KS)):
                    sel = jnp.where(cur[d] == m, d, sel)
                for d in range(KS):
                    cur[d] = jnp.where(sel == d, NEG, cur[d])
                tops.append(m)
                sels.append(sel)
            e1 = jnp.exp((tops[1] - tops[0]) * (1.0 / TEMP))
            e2 = jnp.exp((tops[2] - tops[0]) * (1.0 / TEMP))
            s = 1.0 + e1 + e2
            for w_r, i_r in ((1.0 / s, sels[0]), (e1 / s, sels[1]), (e2 / s, sels[2])):
                plsc.store_scatter(out_v, [row_idx, i_r + off], w_r)
    pltpu.sync_copy(out_v, w_hbm.at[pl.ds(base, 32)])


def _former_body(t_ref, w_ref, ps_ref, pp_ref,
                 wk_ref, bk_ref, n1s_ref, n1b_ref, n2s_ref, n2b_ref,
                 wq_ref, bq_ref, wv_ref, bv_ref, fw1_ref, fb1_ref,
                 fw2_ref, fb2_ref, ns_ref, nb_ref, hw_ref, hb_ref, o_ref):
    x = t_ref[0]
    w = w_ref[0]
    spect = jnp.dot(w[:, :KS], ps_ref[...], preferred_element_type=jnp.float32)
    spat = jnp.dot(w[:, KS:], pp_ref[...], preferred_element_type=jnp.float32)
    wkm = wk_ref[...]
    keys = (jnp.dot(spect, wkm[:BR], preferred_element_type=jnp.float32)
            + jnp.dot(spat, wkm[BR:], preferred_element_type=jnp.float32)
            + bk_ref[...])
    n1s, n1b = n1s_ref[...], n1b_ref[...]
    n2s, n2b = n2s_ref[...], n2b_ref[...]
    wq, bq = wq_ref[...], bq_ref[...]
    wv, bv = wv_ref[...], bv_ref[...]
    fw1, fb1 = fw1_ref[...], fb1_ref[...]
    fw2, fb2 = fw2_ref[...], fb2_ref[...]
    for i in range(DEPTH):
        hh = _ln(x, n1s[i:i + 1], n1b[i:i + 1])
        q = jnp.dot(hh, wq[i], preferred_element_type=jnp.float32) + bq[i:i + 1]
        v = jnp.dot(hh, wv[i], preferred_element_type=jnp.float32) + bv[i:i + 1]
        kv = jnp.dot(keys.T, v, preferred_element_type=jnp.float32)
        attn = jnp.dot(jax.nn.softmax(q, axis=-1), kv,
                       preferred_element_type=jnp.float32)
        x = x + attn
        h2 = _ln(x, n2s[i:i + 1], n2b[i:i + 1])
        g = _gelu(jnp.dot(h2, fw1[i], preferred_element_type=jnp.float32)
                  + fb1[i:i + 1])
        x = x + jnp.dot(g, fw2[i], preferred_element_type=jnp.float32) + fb2[i:i + 1]
    pooled = jnp.mean(x, axis=0, keepdims=True)
    pooled = _ln(pooled, ns_ref[...], nb_ref[...])
    o_ref[0] = jnp.dot(pooled, hw_ref[...], preferred_element_type=jnp.float32) + hb_ref[...]


def kernel(x, band_w, band_b, patch_w, patch_b, spectral_prototypes,
           spatial_prototypes, rw1, rb1, rw2, rb2, wk, bk, n1s, n1b, n2s, n2b,
           wq, bq, wv, bv, fw1, fb1, fw2, fb2, ns, nb, hw, hb):
    B, C, H, W = x.shape
    Hp, Wp = H // P, W // P
    N = Hp * Wp
    bw2 = band_w.reshape(BR, C)
    bb2 = band_b.reshape(BR, 1)
    pw2 = patch_w.transpose(2, 3, 1, 0).reshape(BR * P * P, DIM).astype(jnp.bfloat16)
    pb2 = patch_b.reshape(1, DIM)

    tokens = pl.pallas_call(
        _conv_body,
        grid=(B, Hp),
        in_specs=[
            pl.BlockSpec((1, C, P, W), lambda b, h: (b, 0, h, 0)),
            pl.BlockSpec((BR, C), lambda b, h: (0, 0)),
            pl.BlockSpec((BR, 1), lambda b, h: (0, 0)),
            pl.BlockSpec((BR * P * P, DIM), lambda b, h: (0, 0)),
            pl.BlockSpec((1, DIM), lambda b, h: (0, 0)),
        ],
        out_specs=pl.BlockSpec((1, 1, Wp, DIM), lambda b, h: (b, h, 0, 0)),
        out_shape=jax.ShapeDtypeStruct((B, Hp, Wp, DIM), jnp.float32),
        compiler_params=pltpu.CompilerParams(
            dimension_semantics=("parallel", "arbitrary")),
    )(x, bw2, bb2, pw2, pb2)
    tokens = tokens.reshape(B, N, DIM)

    tok_pad = jnp.concatenate(
        [tokens.reshape(B * N, DIM),
         jnp.zeros((RTOK - B * N, DIM), jnp.float32)], axis=0)
    logits = pl.pallas_call(
        _router_body,
        in_specs=[pl.BlockSpec(s) for s in
                  [(RTOK, DIM), (DIM, 256), (1, 256), (256, KS + KP), (1, KS + KP)]],
        out_specs=pl.BlockSpec((RTOK, KS + KP)),
        out_shape=jax.ShapeDtypeStruct((RTOK, KS + KP), jnp.float32),
    )(tok_pad, rw1, rb1.reshape(1, -1), rw2, rb2.reshape(1, -1))

    route_w = pl.kernel(
        _route_sc,
        out_type=jax.ShapeDtypeStruct((RTOK, KS + KP), jnp.float32),
        mesh=plsc.VectorSubcoreMesh(core_axis_name="c", subcore_axis_name="s"),
        compiler_params=pltpu.CompilerParams(needs_layout_passes=False),
        scratch_types=[
            pltpu.VMEM((32, KS + KP), jnp.float32),
            pltpu.VMEM((32, KS + KP), jnp.float32),
        ],
    )(logits)
    rw3 = route_w[:B * N].reshape(B, N, KS + KP)

    full = lambda a: pl.BlockSpec(a.shape, lambda b: (0,) * a.ndim)
    weights = [
        spectral_prototypes, spatial_prototypes,
        wk, bk.reshape(1, -1), n1s, n1b, n2s, n2b,
        wq, bq, wv, bv, fw1, fb1, fw2, fb2,
        ns.reshape(1, -1), nb.reshape(1, -1), hw, hb.reshape(1, -1),
    ]
    out = pl.pallas_call(
        _former_body,
        grid=(B,),
        in_specs=[pl.BlockSpec((1, N, DIM), lambda b: (b, 0, 0)),
                  pl.BlockSpec((1, N, KS + KP), lambda b: (b, 0, 0))]
                 + [full(a) for a in weights],
        out_specs=pl.BlockSpec((1, 1, 16), lambda b: (b, 0, 0)),
        out_shape=jax.ShapeDtypeStruct((B, 1, 16), jnp.float32),
        compiler_params=pltpu.CompilerParams(
            dimension_semantics=("arbitrary",)),
    )(tokens, rw3, *weights)
    return out.reshape(B, 16)


# speedup vs baseline: 1.0027x; 1.0027x over previous
import functools
import jax
import jax.numpy as jnp
from jax import lax
from jax.experimental import pallas as pl
from jax.experimental.pallas import tpu as pltpu
from jax.experimental.pallas import tpu_sc as plsc

P = 24
DIM = 192
KS = 32
KP = 32
TOPK = 3
TEMP = 0.1
DEPTH = 4
BR = 24
RTOK = 1024
NEG = -3.0e38


def _conv_body(x_ref, bw_ref, bb_ref, pw_ref, pb_ref, o_ref):
    _, C, _, W = x_ref.shape
    Wp = W // P
    xb = x_ref[0].reshape(C, P * W)
    yb = jnp.dot(bw_ref[...], xb, preferred_element_type=jnp.float32)
    yb = yb + bb_ref[...]
    ybT = yb.astype(jnp.bfloat16).T
    t = (ybT.reshape(P, Wp, P, BR)
         .transpose(1, 0, 2, 3).reshape(Wp, P * P * BR))
    out = jnp.dot(t, pw_ref[...], preferred_element_type=jnp.float32) + pb_ref[...]
    o_ref[0, 0] = out


def _gelu(x):
    return 0.5 * x * (1.0 + jax.lax.erf(x * (2.0 ** -0.5)))


def _ln(x, s, b):
    m = x.mean(-1, keepdims=True)
    v = ((x - m) ** 2).mean(-1, keepdims=True)
    return (x - m) / jnp.sqrt(v + 1e-5) * s + b


def _router_body(t_ref, rw1_ref, rb1_ref, rw2_ref, rb2_ref, o_ref):
    n = t_ref.shape[0]
    h = jnp.dot(t_ref[...], rw1_ref[...], preferred_element_type=jnp.float32) + rb1_ref[...]
    h = _gelu(h)
    o_ref[:n, :] = jnp.dot(h, rw2_ref[...], preferred_element_type=jnp.float32) + rb2_ref[...]
    o_ref[n:, :] = jnp.zeros((o_ref.shape[0] - n, o_ref.shape[1]), jnp.float32)


def _route_sc(logits_hbm, w_hbm, chunk_v, out_v):
    wid = lax.axis_index("s") * 2 + lax.axis_index("c")
    base = wid * 32
    pltpu.sync_copy(logits_hbm.at[pl.ds(base, 32)], chunk_v)
    zero = jnp.zeros((16,), jnp.float32)
    for r in range(32):
        for c in range(4):
            out_v[r, pl.ds(c * 16, 16)] = zero
    lane = lax.iota(jnp.int32, 16)
    for g in range(2):
        row_idx = lane + (g * 16)
        vecs = [plsc.load_gather(chunk_v, [row_idx, jnp.full((16,), d, jnp.int32)])
                for d in range(KS + KP)]
        for off in (0, KS):
            cur = [vecs[off + d] for d in range(KS)]
            tops, sels = [], []
            for _ in range(TOPK):
                m = cur[0]
                for d in range(1, KS):
                    m = jnp.maximum(m, cur[d])
                sel = jnp.full((16,), KS, jnp.int32)
                for d in reversed(range(KS)):
                    sel = jnp.where(cur[d] == m, d, sel)
                for d in range(KS):
                    cur[d] = jnp.where(sel == d, NEG, cur[d])
                tops.append(m)
                sels.append(sel)
            e1 = jnp.exp((tops[1] - tops[0]) * (1.0 / TEMP))
            e2 = jnp.exp((tops[2] - tops[0]) * (1.0 / TEMP))
            s = 1.0 + e1 + e2
            for w_r, i_r in ((1.0 / s, sels[0]), (e1 / s, sels[1]), (e2 / s, sels[2])):
                plsc.store_scatter(out_v, [row_idx, i_r + off], w_r)
    pltpu.sync_copy(out_v, w_hbm.at[pl.ds(base, 32)])


def _former_body(t_ref, w_ref, ps_ref, pp_ref,
                 wk_ref, bk_ref, n1s_ref, n1b_ref, n2s_ref, n2b_ref,
                 wq_ref, bq_ref, wv_ref, bv_ref, fw1_ref, fb1_ref,
                 fw2_ref, fb2_ref, ns_ref, nb_ref, hw_ref, hb_ref, o_ref):
    x = t_ref[0]
    w = w_ref[0]
    spect = jnp.dot(w[:, :KS], ps_ref[...], preferred_element_type=jnp.float32)
    spat = jnp.dot(w[:, KS:], pp_ref[...], preferred_element_type=jnp.float32)
    wkm = wk_ref[...]
    keys = (jnp.dot(spect, wkm[:BR], preferred_element_type=jnp.float32)
            + jnp.dot(spat, wkm[BR:], preferred_element_type=jnp.float32)
            + bk_ref[...])
    n1s, n1b = n1s_ref[...], n1b_ref[...]
    n2s, n2b = n2s_ref[...], n2b_ref[...]
    wq, bq = wq_ref[...], bq_ref[...]
    wv, bv = wv_ref[...], bv_ref[...]
    fw1, fb1 = fw1_ref[...], fb1_ref[...]
    fw2, fb2 = fw2_ref[...], fb2_ref[...]
    for i in range(DEPTH):
        hh = _ln(x, n1s[i:i + 1], n1b[i:i + 1])
        q = jnp.dot(hh, wq[i], preferred_element_type=jnp.float32) + bq[i:i + 1]
        v = jnp.dot(hh, wv[i], preferred_element_type=jnp.float32) + bv[i:i + 1]
        kv = jnp.dot(keys.T, v, preferred_element_type=jnp.float32)
        attn = jnp.dot(jax.nn.softmax(q, axis=-1), kv,
                       preferred_element_type=jnp.float32)
        x = x + attn
        h2 = _ln(x, n2s[i:i + 1], n2b[i:i + 1])
        g = _gelu(jnp.dot(h2, fw1[i], preferred_element_type=jnp.float32)
                  + fb1[i:i + 1])
        x = x + jnp.dot(g, fw2[i], preferred_element_type=jnp.float32) + fb2[i:i + 1]
    pooled = jnp.mean(x, axis=0, keepdims=True)
    pooled = _ln(pooled, ns_ref[...], nb_ref[...])
    o_ref[0] = jnp.dot(pooled, hw_ref[...], preferred_element_type=jnp.float32) + hb_ref[...]


def kernel(x, band_w, band_b, patch_w, patch_b, spectral_prototypes,
           spatial_prototypes, rw1, rb1, rw2, rb2, wk, bk, n1s, n1b, n2s, n2b,
           wq, bq, wv, bv, fw1, fb1, fw2, fb2, ns, nb, hw, hb):
    B, C, H, W = x.shape
    Hp, Wp = H // P, W // P
    N = Hp * Wp
    bw2 = band_w.reshape(BR, C)
    bb2 = band_b.reshape(BR, 1)
    pw2 = patch_w.transpose(2, 3, 1, 0).reshape(BR * P * P, DIM).astype(jnp.bfloat16)
    pb2 = patch_b.reshape(1, DIM)

    tokens = pl.pallas_call(
        _conv_body,
        grid=(B, Hp),
        in_specs=[
            pl.BlockSpec((1, C, P, W), lambda b, h: (b, 0, h, 0)),
            pl.BlockSpec((BR, C), lambda b, h: (0, 0)),
            pl.BlockSpec((BR, 1), lambda b, h: (0, 0)),
            pl.BlockSpec((BR * P * P, DIM), lambda b, h: (0, 0)),
            pl.BlockSpec((1, DIM), lambda b, h: (0, 0)),
        ],
        out_specs=pl.BlockSpec((1, 1, Wp, DIM), lambda b, h: (b, h, 0, 0)),
        out_shape=jax.ShapeDtypeStruct((B, Hp, Wp, DIM), jnp.float32),
        compiler_params=pltpu.CompilerParams(
            dimension_semantics=("parallel", "arbitrary")),
    )(x, bw2, bb2, pw2, pb2)
    tokens = tokens.reshape(B, N, DIM)

    logits = pl.pallas_call(
        _router_body,
        in_specs=[pl.BlockSpec(s) for s in
                  [(B * N, DIM), (DIM, 256), (1, 256), (256, KS + KP), (1, KS + KP)]],
        out_specs=pl.BlockSpec((RTOK, KS + KP)),
        out_shape=jax.ShapeDtypeStruct((RTOK, KS + KP), jnp.float32),
    )(tokens.reshape(B * N, DIM), rw1, rb1.reshape(1, -1), rw2, rb2.reshape(1, -1))

    route_w = pl.kernel(
        _route_sc,
        out_type=jax.ShapeDtypeStruct((RTOK, KS + KP), jnp.float32),
        mesh=plsc.VectorSubcoreMesh(core_axis_name="c", subcore_axis_name="s"),
        compiler_params=pltpu.CompilerParams(needs_layout_passes=False),
        scratch_types=[
            pltpu.VMEM((32, KS + KP), jnp.float32),
            pltpu.VMEM((32, KS + KP), jnp.float32),
        ],
    )(logits)

    full = lambda a: pl.BlockSpec(a.shape, lambda b: (0,) * a.ndim)
    weights = [
        spectral_prototypes, spatial_prototypes,
        wk, bk.reshape(1, -1), n1s, n1b, n2s, n2b,
        wq, bq, wv, bv, fw1, fb1, fw2, fb2,
        ns.reshape(1, -1), nb.reshape(1, -1), hw, hb.reshape(1, -1),
    ]
    out = pl.pallas_call(
        _former_body,
        grid=(B,),
        in_specs=[pl.BlockSpec((1, N, DIM), lambda b: (b, 0, 0)),
                  pl.BlockSpec((1, N, KS + KP), lambda b: (b, 0, 0))]
                 + [full(a) for a in weights],
        out_specs=pl.BlockSpec((1, 1, 16), lambda b: (b, 0, 0)),
        out_shape=jax.ShapeDtypeStruct((B, 1, 16), jnp.float32),
        compiler_params=pltpu.CompilerParams(
            dimension_semantics=("arbitrary",)),
    )(tokens, route_w[:B * N].reshape(B, N, KS + KP), *weights)
    return out.reshape(B, 16)
